# baseline (speedup 1.0000x reference)
import jax
import jax.numpy as jnp
from jax import lax
from jax.experimental import pallas as pl
from jax.experimental.pallas import tpu as pltpu
from jax.experimental.pallas import tpu_sc as plsc

B = 16384
NUM_ENT = 100000
NUM_REL = 500
NUM_REL_PAD = 512
NUM_DATE = 4096
NUM_STW = NUM_DATE // 365 + 1
NUM_STW_PAD = 16
S_DIM = 64
T_DIM = 32
R_DIM = 96
CYCLE = 365
NFAC = 1728

NC = 2
NS = 16
LANES = 16
NW = NC * NS
CH = 4096
NB = B // CH

_INV2PI = 0.15915494309189535
_MAGIC = 12582912.0
_C1 = 6.28125
_C2 = 1.9353071795864769e-3
_S0 = 0.9999782156662488
_S1 = -0.16662248279410358
_S2 = 0.008308176673817783
_S3 = -0.00019252550586158768
_S4 = 2.141589485971096e-06


def _psin(x):
    k = (x * _INV2PI + _MAGIC) - _MAGIC
    r = (x - k * _C1) - k * _C2
    t = r * r
    p = _S4 * t + _S3
    p = p * t + _S2
    p = p * t + _S1
    p = p * t + _S0
    return p * r


def _mk_stage_helpers(wid, staged, row_v, idx_bufs, fac_bufs, sem_row, idx_sems,
                      fac_sems):
    def do_table(tbl, n, d, row_base, sets):
        nj = (d + NW - 1) // NW
        subs = [
            (idx_hbm, c, set_off)
            for idx_hbm, set_off in sets
            for c in range(NB)
        ]

        def dim_loop(k, carry):
            j = k * NW + wid

            @pl.when(j < d)
            def _():
                rcp = pltpu.async_copy(tbl.at[j], row_v.at[pl.ds(0, n)], sem_row)
                ih0, c0, _ = subs[0]
                icp = pltpu.async_copy(
                    ih0.at[pl.ds(c0 * CH, CH)], idx_bufs[0], idx_sems[0]
                )
                rcp.wait()
                fcps = [None, None]
                for si, (idx_hbm, c, set_off) in enumerate(subs):
                    icp.wait()
                    if si + 1 < len(subs):
                        nih, nc, _ = subs[si + 1]
                        icp = pltpu.async_copy(
                            nih.at[pl.ds(nc * CH, CH)],
                            idx_bufs[(si + 1) % 2],
                            idx_sems[(si + 1) % 2],
                        )
                    ib = idx_bufs[si % 2]
                    fb = fac_bufs[si % 2]
                    if fcps[si % 2] is not None:
                        fcps[si % 2].wait()

                    @plsc.parallel_loop(0, CH // LANES, 1, unroll=4)
                    def gl(i):
                        sl = pl.ds(i * LANES, LANES)
                        fb[sl] = plsc.load_gather(row_v, [ib[sl]])

                    fcps[si % 2] = pltpu.async_copy(
                        fb,
                        staged.at[row_base + set_off + j, pl.ds(c * CH, CH)],
                        fac_sems[si % 2],
                    )
                for f in fcps:
                    if f is not None:
                        f.wait()

            return carry

        lax.fori_loop(0, nj, dim_loop, 0)

    return do_table


def _stage_big_body(*refs):
    it = iter(refs)
    ent_h = next(it)
    ent_t = next(it)
    rel_f = next(it)
    rel_i = next(it)
    rtc = next(it)
    time_tabs = [next(it) for _ in range(18)]
    heads = next(it)
    tails = next(it)
    rels = next(it)
    dates = next(it)
    staged = next(it)
    row_v = next(it)
    idx_b0 = next(it)
    idx_b1 = next(it)
    fac_b0 = next(it)
    fac_b1 = next(it)
    sem_row = next(it)
    sem_i0 = next(it)
    sem_i1 = next(it)
    sem_f0 = next(it)
    sem_f1 = next(it)

    wid = lax.axis_index("s") * NC + lax.axis_index("c")
    do_table = _mk_stage_helpers(
        wid, staged, row_v, [idx_b0, idx_b1], [fac_b0, fac_b1],
        sem_row, (sem_i0, sem_i1), (sem_f0, sem_f1)
    )

    do_table(ent_h, NUM_ENT, S_DIM, 0, [(heads, 0), (tails, 64)])
    do_table(ent_t, NUM_ENT, S_DIM, 128, [(heads, 0), (tails, 64)])
    do_table(rel_f, NUM_REL_PAD, R_DIM, 256, [(rels, 0)])
    do_table(rel_i, NUM_REL_PAD, R_DIM, 352, [(rels, 0)])
    do_table(rtc, NUM_DATE, R_DIM, 448, [(dates, 0)])
    for t in range(18):
        do_table(
            time_tabs[t], NUM_ENT, T_DIM, 544 + t * T_DIM,
            [(heads, 0), (tails, 576)],
        )


def _stage_small_body(*refs):
    it = iter(refs)
    stw = next(it)
    datesd = next(it)
    staged = next(it)
    row_v = next(it)
    idx_b0 = next(it)
    idx_b1 = next(it)
    fac_b0 = next(it)
    fac_b1 = next(it)
    sem_row = next(it)
    sem_i0 = next(it)
    sem_i1 = next(it)
    sem_f0 = next(it)
    sem_f1 = next(it)

    wid = lax.axis_index("s") * NC + lax.axis_index("c")
    do_table = _mk_stage_helpers(
        wid, staged, row_v, [idx_b0, idx_b1], [fac_b0, fac_b1],
        sem_row, (sem_i0, sem_i1), (sem_f0, sem_f1)
    )

    do_table(stw, NUM_STW_PAD, T_DIM, 0, [(datesd, 0)])


NBIG = 256 + 96 * 3 + 18 * T_DIM * 2
NSMALL = T_DIM

_STAGE_SCRATCH = [
    pltpu.VMEM((NUM_ENT,), jnp.float32),
    pltpu.VMEM((CH,), jnp.int32),
    pltpu.VMEM((CH,), jnp.int32),
    pltpu.VMEM((CH,), jnp.float32),
    pltpu.VMEM((CH,), jnp.float32),
    pltpu.SemaphoreType.DMA,
    pltpu.SemaphoreType.DMA,
    pltpu.SemaphoreType.DMA,
    pltpu.SemaphoreType.DMA,
    pltpu.SemaphoreType.DMA,
]

_SC_MESH = plsc.VectorSubcoreMesh(core_axis_name="c", subcore_axis_name="s")

_stage_big_call = pl.kernel(
    _stage_big_body,
    out_type=jax.ShapeDtypeStruct((NBIG, B), jnp.float32),
    mesh=_SC_MESH,
    scratch_types=_STAGE_SCRATCH,
    compiler_params=pltpu.CompilerParams(
        needs_layout_passes=False, use_tc_tiling_on_sc=True
    ),
)

_stage_small_call = pl.kernel(
    _stage_small_body,
    out_type=jax.ShapeDtypeStruct((NSMALL, B), jnp.float32),
    mesh=_SC_MESH,
    scratch_types=_STAGE_SCRATCH,
    compiler_params=pltpu.CompilerParams(
        needs_layout_passes=False, use_tc_tiling_on_sc=False
    ),
)

_CBLK = 1024


def _combine_body(stb, sts, yr_r, mo_r, dy_r, out_r):
    yr = yr_r[...]
    mo = mo_r[...]
    dy = dy_r[...]

    acc = jnp.zeros((_CBLK,), jnp.float32)
    for j in range(S_DIM):
        h1 = stb[0 + j, :]
        h2 = stb[64 + j, :]
        t2 = stb[128 + j, :]
        t1 = stb[192 + j, :]
        rf = stb[256 + j, :]
        ri = stb[352 + j, :]
        tm = stb[448 + j, :]
        acc = acc + h1 * (rf + rf * tm) * t1 + h2 * (ri + ri * tm) * t2

    def tte(s, pfx, j):
        v = sts[j, :]
        for per, tv in ((0, yr), (1, mo), (2, dy)):
            fq = stb[544 + s * 576 + (pfx * 9 + 0 * 3 + per) * T_DIM + j, :]
            ph = stb[544 + s * 576 + (pfx * 9 + 1 * 3 + per) * T_DIM + j, :]
            am = stb[544 + s * 576 + (pfx * 9 + 2 * 3 + per) * T_DIM + j, :]
            v = v + am * _psin(fq * tv + ph)
        return v

    for j in range(T_DIM):
        rf = stb[256 + S_DIM + j, :]
        ri = stb[352 + S_DIM + j, :]
        tm = stb[448 + S_DIM + j, :]
        h1 = tte(0, 0, j)
        t1 = tte(1, 1, j)
        h2 = tte(1, 0, j)
        t2 = tte(0, 1, j)
        acc = acc + h1 * (rf + rf * tm) * t1 + h2 * (ri + ri * tm) * t2

    out_r[...] = acc * 0.5


def _combine_call(staged_big, staged_small, yrf, mof, dyf):
    grid = (B // _CBLK,)
    vec = pl.BlockSpec((_CBLK,), lambda i: (i,))
    return pl.pallas_call(
        _combine_body,
        grid=grid,
        in_specs=[
            pl.BlockSpec((NBIG, _CBLK), lambda i: (0, i)),
            pl.BlockSpec((NSMALL, _CBLK), lambda i: (0, i)),
            vec,
            vec,
            vec,
        ],
        out_specs=vec,
        out_shape=jax.ShapeDtypeStruct((B,), jnp.float32),
    )(staged_big, staged_small, yrf, mof, dyf)


def kernel(params, heads, rels, tails, years, months, days, date_ids):
    heads = heads.astype(jnp.int32)
    tails = tails.astype(jnp.int32)
    rels = rels.astype(jnp.int32)
    dates = date_ids.astype(jnp.int32)
    datesd = (date_ids // CYCLE).astype(jnp.int32)
    yrf = years.astype(jnp.float32)
    mof = months.astype(jnp.float32)
    dyf = days.astype(jnp.float32)

    time_tabs = [
        params[f"{per}_{kind}_{pfx}"].T
        for pfx in ("h", "t")
        for kind in ("freq", "phi", "amps")
        for per in ("y", "m", "d")
    ]
    rel_f_p = jnp.pad(params["rel_embs_f"], ((0, NUM_REL_PAD - NUM_REL), (0, 0)))
    rel_i_p = jnp.pad(params["rel_embs_i"], ((0, NUM_REL_PAD - NUM_REL), (0, 0)))
    stw_p = jnp.pad(params["stw"], ((0, NUM_STW_PAD - NUM_STW), (0, 0)))
    staged_small = _stage_small_call(
        stw_p.T,
        datesd,
    )
    staged_big = _stage_big_call(
        params["ent_embs_h"].T,
        params["ent_embs_t"].T,
        rel_f_p.T,
        rel_i_p.T,
        params["rtc"].T,
        *time_tabs,
        heads,
        tails,
        rels,
        dates,
    )
    return _combine_call(staged_big, staged_small, yrf, mof, dyf)
